# baseline (speedup 1.0000x reference)
import jax
import jax.numpy as jnp
from jax.experimental import pallas as pl
from jax.experimental.pallas import tpu as pltpu

_MARGIN = 1.0
_LAMBDA = 1.0
_TILE_PAIRS = 256


def _normalize_body(x_ref, o_ref):
    x = x_ref[...]
    nsq = jnp.sum(x * x, axis=1, keepdims=True)
    o_ref[...] = x * jax.lax.rsqrt(jnp.maximum(nsq, 1e-16))


def _pair_loss_body(idx_hbm, tab_hbm, out_ref,
                    tab_vmem, prod_vmem, acc_vmem, idx_smem,
                    idx_sem, tab_sem, *,
                    tile_pairs, num_inner, pos_tiles, w_pos, w_neg,
                    super_tiles, num_super):
    o = pl.program_id(0)
    i = pl.program_id(1)
    t = 2 * i + o
    s = jax.lax.div(i, super_tiles)
    j = jax.lax.rem(i, super_tiles)
    slot = jax.lax.rem(s, 2)

    @pl.when(i == 0)
    def _prologue():
        pltpu.make_async_copy(tab_hbm, tab_vmem, tab_sem).start()
        pltpu.make_async_copy(idx_hbm.at[o, 0], idx_smem.at[0],
                              idx_sem.at[0]).start()
        pltpu.make_async_copy(tab_hbm, tab_vmem, tab_sem).wait()
        acc_vmem[...] = jnp.zeros_like(acc_vmem)

    @pl.when((j == 0) & (s + 1 < num_super))
    def _prefetch_next():
        slot_next = jax.lax.rem(s + 1, 2)
        pltpu.make_async_copy(idx_hbm.at[o, s + 1], idx_smem.at[slot_next],
                              idx_sem.at[slot_next]).start()

    @pl.when(j == 0)
    def _wait_super():
        pltpu.make_async_copy(idx_hbm.at[o, s], idx_smem.at[slot],
                              idx_sem.at[slot]).wait()

    base = j * (2 * tile_pairs)
    for mi in range(tile_pairs):
        i1 = idx_smem[slot, base + mi]
        i2 = idx_smem[slot, base + tile_pairs + mi]
        r1 = tab_vmem[i1, 0, :]
        r2 = tab_vmem[i2, 0, :]
        prod_vmem[mi, :] = r1 * r2

    dots = jnp.sum(prod_vmem[...], axis=1, keepdims=True)
    dist = 1.0 - dots
    hinge = jnp.maximum(_MARGIN - dist, 0.0)
    is_neg = t >= pos_tiles
    contrib = jnp.where(is_neg, w_neg * hinge, w_pos * dist)
    acc_vmem[...] += contrib

    @pl.when(i == num_inner - 1)
    def _finalize():
        out_ref[...] = jnp.zeros((1, 1, 128), jnp.float32) \
            + jnp.sum(acc_vmem[...])


def _normalize(embeddings):
    n, d = (int(s) for s in embeddings.shape)
    rows = 5000 if n % 10000 == 0 else 8
    grid_inner = n // (2 * rows)
    assert n % (2 * rows) == 0
    return pl.pallas_call(
        _normalize_body,
        out_shape=jax.ShapeDtypeStruct((n, d), jnp.float32),
        grid=(2, grid_inner),
        in_specs=[pl.BlockSpec((rows, d), lambda o, i: (o * (n // (2 * rows)) + i, 0))],
        out_specs=pl.BlockSpec((rows, d), lambda o, i: (o * (n // (2 * rows)) + i, 0)),
        compiler_params=pltpu.CompilerParams(
            dimension_semantics=("parallel", "arbitrary")),
    )(embeddings)


def kernel(embeddings, positive_pairs, negative_pairs):
    num_nodes, emb_dim = (int(s) for s in embeddings.shape)
    num_pos = int(positive_pairs.shape[0])
    num_neg = int(negative_pairs.shape[0])
    tp = _TILE_PAIRS
    assert num_pos % tp == 0 and num_neg % tp == 0

    pos_tiles = num_pos // tp
    num_tiles = pos_tiles + num_neg // tp
    assert num_tiles % 2 == 0
    num_inner = num_tiles // 2

    unit = _normalize(embeddings).reshape(num_nodes, 1, emb_dim)

    pairs = jnp.concatenate([positive_pairs.astype(jnp.int32),
                             negative_pairs.astype(jnp.int32)], axis=0)
    sup = 8
    assert num_inner % sup == 0
    num_super = num_inner // sup
    idx = pairs.reshape(num_tiles, tp, 2).transpose(0, 2, 1) \
               .reshape(num_tiles, 2 * tp)
    idx = jnp.stack([idx[0::2], idx[1::2]]) \
             .reshape(2, num_super, sup * 2 * tp)

    partials = pl.pallas_call(
        lambda *refs: _pair_loss_body(
            *refs, tile_pairs=tp, num_inner=num_inner, pos_tiles=pos_tiles,
            w_pos=1.0 / num_pos, w_neg=1.0 / num_neg,
            super_tiles=sup, num_super=num_super),
        out_shape=jax.ShapeDtypeStruct((2, 1, 128), jnp.float32),
        grid_spec=pltpu.PrefetchScalarGridSpec(
            num_scalar_prefetch=0,
            grid=(2, num_inner),
            in_specs=[
                pl.BlockSpec(memory_space=pl.ANY),
                pl.BlockSpec(memory_space=pl.ANY),
            ],
            out_specs=pl.BlockSpec((1, 1, 128), lambda o, i: (o, 0, 0)),
            scratch_shapes=[
                pltpu.VMEM((num_nodes, 1, emb_dim), jnp.float32),
                pltpu.VMEM((tp, emb_dim), jnp.float32),
                pltpu.VMEM((tp, 1), jnp.float32),
                pltpu.SMEM((2, sup * 2 * tp), jnp.int32),
                pltpu.SemaphoreType.DMA((2,)),
                pltpu.SemaphoreType.DMA,
            ]),
        compiler_params=pltpu.CompilerParams(
            dimension_semantics=("parallel", "arbitrary"),
            vmem_limit_bytes=64 * 1024 * 1024),
    )(idx, unit)

    return _LAMBDA * jnp.sum(partials[:, 0, 0])


# speedup vs baseline: 1.9113x; 1.9113x over previous
import jax
import jax.numpy as jnp
from jax.experimental import pallas as pl
from jax.experimental.pallas import tpu as pltpu

_MARGIN = 1.0
_LAMBDA = 1.0
_TILE_PAIRS = 256


def _normalize_body(x_ref, o_ref):
    x = x_ref[...]
    nsq = jnp.sum(x * x, axis=1, keepdims=True)
    o_ref[...] = x * jax.lax.rsqrt(jnp.maximum(nsq, 1e-16))


def _pair_loss_body(idx_hbm, tab_hbm, out_ref,
                    tab_vmem, prod_vmem, acc_vmem, idx_smem,
                    idx_sem, tab_sem, *,
                    tile_pairs, num_inner, pos_tiles, w_pos, w_neg,
                    super_tiles, num_super):
    o = pl.program_id(0)
    i = pl.program_id(1)
    t = 2 * i + o
    s = jax.lax.div(i, super_tiles)
    j = jax.lax.rem(i, super_tiles)
    slot = jax.lax.rem(s, 2)

    @pl.when(i == 0)
    def _prologue():
        pltpu.make_async_copy(tab_hbm, tab_vmem, tab_sem).start()
        pltpu.make_async_copy(idx_hbm.at[o, 0], idx_smem.at[0],
                              idx_sem.at[0]).start()
        pltpu.make_async_copy(tab_hbm, tab_vmem, tab_sem).wait()
        acc_vmem[...] = jnp.zeros_like(acc_vmem)

    @pl.when((j == 0) & (s + 1 < num_super))
    def _prefetch_next():
        slot_next = jax.lax.rem(s + 1, 2)
        pltpu.make_async_copy(idx_hbm.at[o, s + 1], idx_smem.at[slot_next],
                              idx_sem.at[slot_next]).start()

    @pl.when(j == 0)
    def _wait_super():
        pltpu.make_async_copy(idx_hbm.at[o, s], idx_smem.at[slot],
                              idx_sem.at[slot]).wait()

    for mi in range(tile_pairs):
        i1 = idx_smem[slot, j, mi]
        i2 = idx_smem[slot, j, tile_pairs + mi]
        r1 = tab_vmem[i1, 0, :]
        r2 = tab_vmem[i2, 0, :]
        prod_vmem[mi, :] = r1 * r2

    dots = jnp.sum(prod_vmem[...], axis=1, keepdims=True)
    dist = 1.0 - dots
    hinge = jnp.maximum(_MARGIN - dist, 0.0)
    is_neg = t >= pos_tiles
    contrib = jnp.where(is_neg, w_neg * hinge, w_pos * dist)
    acc_vmem[...] += contrib

    @pl.when(i == num_inner - 1)
    def _finalize():
        out_ref[...] = jnp.zeros((1, 1, 128), jnp.float32) \
            + jnp.sum(acc_vmem[...])


def _normalize(embeddings):
    n, d = (int(s) for s in embeddings.shape)
    rows = 5000 if n % 10000 == 0 else 8
    grid_inner = n // (2 * rows)
    assert n % (2 * rows) == 0
    return pl.pallas_call(
        _normalize_body,
        out_shape=jax.ShapeDtypeStruct((n, d), jnp.float32),
        grid=(2, grid_inner),
        in_specs=[pl.BlockSpec((rows, d), lambda o, i: (o * (n // (2 * rows)) + i, 0))],
        out_specs=pl.BlockSpec((rows, d), lambda o, i: (o * (n // (2 * rows)) + i, 0)),
        compiler_params=pltpu.CompilerParams(
            dimension_semantics=("parallel", "arbitrary")),
    )(embeddings)


def kernel(embeddings, positive_pairs, negative_pairs):
    num_nodes, emb_dim = (int(s) for s in embeddings.shape)
    num_pos = int(positive_pairs.shape[0])
    num_neg = int(negative_pairs.shape[0])
    tp = _TILE_PAIRS
    assert num_pos % tp == 0 and num_neg % tp == 0

    pos_tiles = num_pos // tp
    num_tiles = pos_tiles + num_neg // tp
    assert num_tiles % 2 == 0
    num_inner = num_tiles // 2

    unit = _normalize(embeddings).reshape(num_nodes, 1, emb_dim)

    pairs = jnp.concatenate([positive_pairs.astype(jnp.int32),
                             negative_pairs.astype(jnp.int32)], axis=0)
    sup = 8
    assert num_inner % sup == 0
    num_super = num_inner // sup
    idx = pairs.reshape(num_tiles, tp, 2).transpose(0, 2, 1) \
               .reshape(num_tiles, 2 * tp)
    idx = jnp.stack([idx[0::2], idx[1::2]]) \
             .reshape(2, num_super, sup, 2 * tp)

    partials = pl.pallas_call(
        lambda *refs: _pair_loss_body(
            *refs, tile_pairs=tp, num_inner=num_inner, pos_tiles=pos_tiles,
            w_pos=1.0 / num_pos, w_neg=1.0 / num_neg,
            super_tiles=sup, num_super=num_super),
        out_shape=jax.ShapeDtypeStruct((2, 1, 128), jnp.float32),
        grid_spec=pltpu.PrefetchScalarGridSpec(
            num_scalar_prefetch=0,
            grid=(2, num_inner),
            in_specs=[
                pl.BlockSpec(memory_space=pl.ANY),
                pl.BlockSpec(memory_space=pl.ANY),
            ],
            out_specs=pl.BlockSpec((1, 1, 128), lambda o, i: (o, 0, 0)),
            scratch_shapes=[
                pltpu.VMEM((num_nodes, 1, emb_dim), jnp.float32),
                pltpu.VMEM((tp, emb_dim), jnp.float32),
                pltpu.VMEM((tp, 1), jnp.float32),
                pltpu.SMEM((2, sup, 2 * tp), jnp.int32),
                pltpu.SemaphoreType.DMA((2,)),
                pltpu.SemaphoreType.DMA,
            ]),
        compiler_params=pltpu.CompilerParams(
            dimension_semantics=("parallel", "arbitrary"),
            vmem_limit_bytes=64 * 1024 * 1024),
    )(idx, unit)

    return _LAMBDA * jnp.sum(partials[:, 0, 0])


# speedup vs baseline: 2.2905x; 1.1984x over previous
import jax
import jax.numpy as jnp
from jax.experimental import pallas as pl
from jax.experimental.pallas import tpu as pltpu

_MARGIN = 1.0
_LAMBDA = 1.0
_TILE_PAIRS = 2048
_TABLE_CHUNKS = 8


def _pair_loss_body(idx_hbm, tab_hbm, out_ref,
                    tab_vmem, prod_vmem, acc_vmem, idx_sm_a, idx_sm_b,
                    idx_sem, tab_sem, *,
                    tile_pairs, num_inner, pos_tiles, w_pos, w_neg,
                    num_nodes):
    o = pl.program_id(0)
    i = pl.program_id(1)
    t = o * num_inner + i
    parity = jax.lax.rem(i, 2)

    nc = _TABLE_CHUNKS
    ch = num_nodes // nc
    sub = 500 if ch % 500 == 0 else ch

    @pl.when(i == 0)
    def _prologue():
        for c in range(nc):
            sl = pl.ds(c * ch, ch)
            pltpu.make_async_copy(tab_hbm.at[sl], tab_vmem.at[sl],
                                  tab_sem.at[c]).start()
        pltpu.make_async_copy(idx_hbm.at[o, 0], idx_sm_a,
                              idx_sem.at[0]).start()
        for c in range(nc):
            sl = pl.ds(c * ch, ch)
            pltpu.make_async_copy(tab_hbm.at[sl], tab_vmem.at[sl],
                                  tab_sem.at[c]).wait()
            base = c * ch

            def _norm_block(s2, _):
                bsl = pl.ds(base + s2 * sub, sub)
                x = tab_vmem[bsl]
                nsq = jnp.sum(x * x, axis=2, keepdims=True)
                tab_vmem[bsl] = x * jax.lax.rsqrt(jnp.maximum(nsq, 1e-16))
                return 0

            jax.lax.fori_loop(0, ch // sub, _norm_block, 0)
        acc_vmem[...] = jnp.zeros_like(acc_vmem)

    @pl.when((i + 1 < num_inner) & (parity == 0))
    def _prefetch_b():
        pltpu.make_async_copy(idx_hbm.at[o, i + 1], idx_sm_b,
                              idx_sem.at[1]).start()

    @pl.when((i + 1 < num_inner) & (parity == 1))
    def _prefetch_a():
        pltpu.make_async_copy(idx_hbm.at[o, i + 1], idx_sm_a,
                              idx_sem.at[0]).start()

    def _do_tile(idx_sm, sem):
        pltpu.make_async_copy(idx_hbm.at[o, i], idx_sm, sem).wait()
        for mi in range(tile_pairs):
            i1 = idx_sm[mi]
            i2 = idx_sm[tile_pairs + mi]
            r1 = tab_vmem[i1, 0, :]
            r2 = tab_vmem[i2, 0, :]
            prod_vmem[mi, :] = r1 * r2

        dots = jnp.sum(prod_vmem[...], axis=1, keepdims=True)
        dist = 1.0 - dots
        hinge = jnp.maximum(_MARGIN - dist, 0.0)
        is_neg = t >= pos_tiles
        contrib = jnp.where(is_neg, w_neg * hinge, w_pos * dist)
        acc_vmem[...] += contrib

    @pl.when(parity == 0)
    def _tile_a():
        _do_tile(idx_sm_a, idx_sem.at[0])

    @pl.when(parity == 1)
    def _tile_b():
        _do_tile(idx_sm_b, idx_sem.at[1])

    @pl.when(i == num_inner - 1)
    def _finalize():
        out_ref[...] = jnp.zeros((1, 1, 128), jnp.float32) \
            + jnp.sum(acc_vmem[...])


def kernel(embeddings, positive_pairs, negative_pairs):
    num_nodes, emb_dim = (int(s) for s in embeddings.shape)
    num_pos = int(positive_pairs.shape[0])
    num_neg = int(negative_pairs.shape[0])
    tp = _TILE_PAIRS
    assert num_pos % tp == 0 and num_neg % tp == 0
    assert num_nodes % _TABLE_CHUNKS == 0

    pos_tiles = num_pos // tp
    num_tiles = pos_tiles + num_neg // tp
    assert num_tiles % 2 == 0
    num_inner = num_tiles // 2

    table = embeddings.reshape(num_nodes, 1, emb_dim)

    pairs = jnp.concatenate([positive_pairs.astype(jnp.int32),
                             negative_pairs.astype(jnp.int32)], axis=0)
    idx = pairs.reshape(num_tiles, tp, 2).transpose(0, 2, 1) \
               .reshape(2, num_inner, 2 * tp)

    partials = pl.pallas_call(
        lambda *refs: _pair_loss_body(
            *refs, tile_pairs=tp, num_inner=num_inner, pos_tiles=pos_tiles,
            w_pos=1.0 / num_pos, w_neg=1.0 / num_neg, num_nodes=num_nodes),
        out_shape=jax.ShapeDtypeStruct((2, 1, 128), jnp.float32),
        grid_spec=pltpu.PrefetchScalarGridSpec(
            num_scalar_prefetch=0,
            grid=(2, num_inner),
            in_specs=[
                pl.BlockSpec(memory_space=pl.ANY),
                pl.BlockSpec(memory_space=pl.ANY),
            ],
            out_specs=pl.BlockSpec((1, 1, 128), lambda o, i: (o, 0, 0)),
            scratch_shapes=[
                pltpu.VMEM((num_nodes, 1, emb_dim), jnp.float32),
                pltpu.VMEM((tp, emb_dim), jnp.float32),
                pltpu.VMEM((tp, 1), jnp.float32),
                pltpu.SMEM((2 * tp,), jnp.int32),
                pltpu.SMEM((2 * tp,), jnp.int32),
                pltpu.SemaphoreType.DMA((2,)),
                pltpu.SemaphoreType.DMA((_TABLE_CHUNKS,)),
            ]),
        compiler_params=pltpu.CompilerParams(
            dimension_semantics=("parallel", "arbitrary"),
            vmem_limit_bytes=64 * 1024 * 1024),
    )(idx, table)

    return _LAMBDA * jnp.sum(partials[:, 0, 0])


# speedup vs baseline: 3.2375x; 1.4134x over previous
import jax
import jax.numpy as jnp
from jax.experimental import pallas as pl
from jax.experimental.pallas import tpu as pltpu

_MARGIN = 1.0
_LAMBDA = 1.0
_TILE_PAIRS = 2048


def _normalize_body(x_ref, o_ref):
    x = x_ref[...]
    nsq = jnp.sum(x * x, axis=1, keepdims=True)
    o_ref[...] = x * jax.lax.rsqrt(jnp.maximum(nsq, 1e-16))


def _pair_loss_body(idx_hbm, tab_hbm, out_ref,
                    tab_vmem, prod_vmem, acc_vmem, idx_sm_a, idx_sm_b,
                    idx_sem, tab_sem, *,
                    tile_pairs, num_inner, pos_tiles, w_pos, w_neg):
    o = pl.program_id(0)
    i = pl.program_id(1)
    t = o * num_inner + i
    parity = jax.lax.rem(i, 2)

    @pl.when(i == 0)
    def _prologue():
        pltpu.make_async_copy(tab_hbm, tab_vmem, tab_sem).start()
        pltpu.make_async_copy(idx_hbm.at[o, 0], idx_sm_a,
                              idx_sem.at[0]).start()
        pltpu.make_async_copy(tab_hbm, tab_vmem, tab_sem).wait()
        acc_vmem[...] = jnp.zeros_like(acc_vmem)

    @pl.when((i + 1 < num_inner) & (parity == 0))
    def _prefetch_b():
        pltpu.make_async_copy(idx_hbm.at[o, i + 1], idx_sm_b,
                              idx_sem.at[1]).start()

    @pl.when((i + 1 < num_inner) & (parity == 1))
    def _prefetch_a():
        pltpu.make_async_copy(idx_hbm.at[o, i + 1], idx_sm_a,
                              idx_sem.at[0]).start()

    def _do_tile(idx_sm, sem):
        pltpu.make_async_copy(idx_hbm.at[o, i], idx_sm, sem).wait()
        for mi in range(tile_pairs):
            i1 = idx_sm[mi]
            i2 = idx_sm[tile_pairs + mi]
            r1 = tab_vmem[i1, 0, :]
            r2 = tab_vmem[i2, 0, :]
            prod_vmem[mi, :] = r1 * r2

        dots = jnp.sum(prod_vmem[...], axis=1, keepdims=True)
        dist = 1.0 - dots
        hinge = jnp.maximum(_MARGIN - dist, 0.0)
        is_neg = t >= pos_tiles
        contrib = jnp.where(is_neg, w_neg * hinge, w_pos * dist)
        acc_vmem[...] += contrib

    @pl.when(parity == 0)
    def _tile_a():
        _do_tile(idx_sm_a, idx_sem.at[0])

    @pl.when(parity == 1)
    def _tile_b():
        _do_tile(idx_sm_b, idx_sem.at[1])

    @pl.when(i == num_inner - 1)
    def _finalize():
        out_ref[...] = jnp.zeros((1, 1, 128), jnp.float32) \
            + jnp.sum(acc_vmem[...])


def _normalize(embeddings):
    n, d = (int(s) for s in embeddings.shape)
    rows = 5000 if n % 10000 == 0 else 8
    grid_inner = n // (2 * rows)
    assert n % (2 * rows) == 0
    return pl.pallas_call(
        _normalize_body,
        out_shape=jax.ShapeDtypeStruct((n, d), jnp.float32),
        grid=(2, grid_inner),
        in_specs=[pl.BlockSpec((rows, d), lambda o, i: (o * (n // (2 * rows)) + i, 0))],
        out_specs=pl.BlockSpec((rows, d), lambda o, i: (o * (n // (2 * rows)) + i, 0)),
        compiler_params=pltpu.CompilerParams(
            dimension_semantics=("parallel", "arbitrary")),
    )(embeddings)


def kernel(embeddings, positive_pairs, negative_pairs):
    num_nodes, emb_dim = (int(s) for s in embeddings.shape)
    num_pos = int(positive_pairs.shape[0])
    num_neg = int(negative_pairs.shape[0])
    tp = _TILE_PAIRS
    assert num_pos % tp == 0 and num_neg % tp == 0

    pos_tiles = num_pos // tp
    num_tiles = pos_tiles + num_neg // tp
    assert num_tiles % 2 == 0
    num_inner = num_tiles // 2

    unit = _normalize(embeddings).reshape(num_nodes, 1, emb_dim)

    pairs = jnp.concatenate([positive_pairs.astype(jnp.int32),
                             negative_pairs.astype(jnp.int32)], axis=0)
    idx = pairs.reshape(num_tiles, tp, 2).transpose(0, 2, 1) \
               .reshape(2, num_inner, 2 * tp)

    partials = pl.pallas_call(
        lambda *refs: _pair_loss_body(
            *refs, tile_pairs=tp, num_inner=num_inner, pos_tiles=pos_tiles,
            w_pos=1.0 / num_pos, w_neg=1.0 / num_neg),
        out_shape=jax.ShapeDtypeStruct((2, 1, 128), jnp.float32),
        grid_spec=pltpu.PrefetchScalarGridSpec(
            num_scalar_prefetch=0,
            grid=(2, num_inner),
            in_specs=[
                pl.BlockSpec(memory_space=pl.ANY),
                pl.BlockSpec(memory_space=pl.ANY),
            ],
            out_specs=pl.BlockSpec((1, 1, 128), lambda o, i: (o, 0, 0)),
            scratch_shapes=[
                pltpu.VMEM((num_nodes, 1, emb_dim), jnp.float32),
                pltpu.VMEM((tp, emb_dim), jnp.float32),
                pltpu.VMEM((tp, 1), jnp.float32),
                pltpu.SMEM((2 * tp,), jnp.int32),
                pltpu.SMEM((2 * tp,), jnp.int32),
                pltpu.SemaphoreType.DMA((2,)),
                pltpu.SemaphoreType.DMA,
            ]),
        compiler_params=pltpu.CompilerParams(
            dimension_semantics=("parallel", "arbitrary"),
            vmem_limit_bytes=64 * 1024 * 1024),
    )(idx, unit)

    return _LAMBDA * jnp.sum(partials[:, 0, 0])
